# baseline (speedup 1.0000x reference)
import functools
import jax
import jax.numpy as jnp
from jax import lax
from jax.experimental import pallas as pl
from jax.experimental.pallas import tpu as pltpu
from jax.experimental.pallas import tpu_sc as plsc

NC = 2
NS = 16
LANES = 16


def kernel(x, pos_table):
    B, S, D = x.shape
    NW = NC * NS
    rows_per_w = S // NW
    R = 64
    n_chunks = rows_per_w // R

    mesh = plsc.VectorSubcoreMesh(
        core_axis_name="c", subcore_axis_name="s", num_cores=NC, num_subcores=NS
    )

    @functools.partial(
        pl.kernel,
        mesh=mesh,
        out_type=jax.ShapeDtypeStruct((B, S, D), x.dtype),
        scratch_types=[
            pltpu.VMEM((R, D), jnp.float32),
            pltpu.VMEM((R, D), jnp.float32),
        ],
    )
    def sc_add(x_hbm, pos_hbm, out_hbm, pbuf, xbuf):
        wid = lax.axis_index("s") * NC + lax.axis_index("c")
        base = wid * rows_per_w

        def chunk_body(c, carry):
            s0 = base + c * R
            pltpu.sync_copy(pos_hbm.at[pl.ds(s0, R), :], pbuf)
            for b in range(B):
                pltpu.sync_copy(x_hbm.at[b, pl.ds(s0, R), :], xbuf)

                def row_body(r, rcarry):
                    for j in range(D // LANES):
                        sl = pl.ds(j * LANES, LANES)
                        xbuf[r, sl] = xbuf[r, sl] + pbuf[r, sl]
                    return rcarry

                lax.fori_loop(0, R, row_body, 0)
                pltpu.sync_copy(xbuf, out_hbm.at[b, pl.ds(s0, R), :])
            return carry

        lax.fori_loop(0, n_chunks, chunk_body, 0)

    return sc_add(x, pos_table)
